# baseline (speedup 1.0000x reference)
import functools

import jax
import jax.numpy as jnp
from jax.experimental import pallas as pl
from jax.experimental.pallas import tpu as pltpu

N = 10000
E = 320000
D_NODE = 128
D_EDGE = 16
HID = 256
NH = 8
HEAD = HID // NH

CLF_TILE = 2560


def _leaky(x):
    return jnp.where(x > 0, x, 0.2 * x)


def _clf_body(gs_ref, gd_ref, ea_ref, we_ref, b1_ref, w2_ref, b2_ref,
              w3_ref, b3_ref, out_ref):
    a = (gs_ref[...] + gd_ref[...]
         + jnp.dot(ea_ref[...], we_ref[...], preferred_element_type=jnp.float32)
         + b1_ref[...])
    h1 = jnp.maximum(a, 0.0)
    h2 = jnp.maximum(
        jnp.dot(h1, w2_ref[...], preferred_element_type=jnp.float32) + b2_ref[...],
        0.0)
    out_ref[...] = (
        jnp.dot(h2, w3_ref[...], preferred_element_type=jnp.float32) + b3_ref[...])


@jax.jit
def _classifier(gs, gd, ea, we, b1, w2, b2, w3, b3):
    grid = (E // CLF_TILE,)
    return pl.pallas_call(
        _clf_body,
        grid=grid,
        in_specs=[
            pl.BlockSpec((CLF_TILE, HID), lambda i: (i, 0)),
            pl.BlockSpec((CLF_TILE, HID), lambda i: (i, 0)),
            pl.BlockSpec((CLF_TILE, D_EDGE), lambda i: (i, 0)),
            pl.BlockSpec((D_EDGE, 256), lambda i: (0, 0)),
            pl.BlockSpec((1, 256), lambda i: (0, 0)),
            pl.BlockSpec((256, 128), lambda i: (0, 0)),
            pl.BlockSpec((1, 128), lambda i: (0, 0)),
            pl.BlockSpec((128, 2), lambda i: (0, 0)),
            pl.BlockSpec((1, 2), lambda i: (0, 0)),
        ],
        out_specs=pl.BlockSpec((CLF_TILE, 2), lambda i: (i, 0)),
        out_shape=jax.ShapeDtypeStruct((E, 2), jnp.float32),
    )(gs, gd, ea, we, b1, w2, b2, w3, b3)


def _gat_layer(x, src, dst, W, a_s, a_d, b, concat):
    n = x.shape[0]
    h = (x @ W).reshape(n, a_s.shape[0], a_s.shape[1])
    es = jnp.sum(h * a_s[None, :, :], axis=-1)
    ed = jnp.sum(h * a_d[None, :, :], axis=-1)
    e = _leaky(es[src] + ed[dst])
    m = jax.ops.segment_max(e, dst, num_segments=n)
    m = jnp.where(jnp.isfinite(m), m, 0.0)
    ex = jnp.exp(e - m[dst])
    den = jax.ops.segment_sum(ex, dst, num_segments=n)
    alpha = ex / (den[dst] + 1e-16)
    msg = h[src] * alpha[:, :, None]
    out = jax.ops.segment_sum(msg, dst, num_segments=n)
    if concat:
        out = out.reshape(n, -1)
    else:
        out = out.mean(axis=1)
    return out + b


def kernel(node_features, edge_index, edge_attr, W_enc, b_enc, W0, a_s0, a_d0,
           b0, W1, a_s1, a_d1, b1, W2, a_s2, a_d2, b2, Wc1, bc1, g1, be1, Wc2,
           bc2, g2, be2, Wc3, bc3):
    src, dst = edge_index[0], edge_index[1]
    x = jax.nn.relu(node_features @ W_enc + b_enc)
    x = jax.nn.elu(_gat_layer(x, src, dst, W0, a_s0, a_d0, b0, True))
    x = jax.nn.elu(_gat_layer(x, src, dst, W1, a_s1, a_d1, b1, True))
    x = jax.nn.elu(_gat_layer(x, src, dst, W2, a_s2, a_d2, b2, False))

    bn_scale = 1.0 / jnp.sqrt(1.0 + 1e-5)
    s1 = bn_scale * g1
    w1s = Wc1 * s1[None, :]
    beta1 = bc1 * s1 + be1
    s2 = bn_scale * g2
    w2s = Wc2 * s2[None, :]
    beta2 = bc2 * s2 + be2

    ps = x @ w1s[:HID]
    pd = x @ w1s[HID:2 * HID]
    gs = ps[src]
    gd = pd[dst]
    logits = _classifier(gs, gd, edge_attr, w1s[2 * HID:],
                         beta1.reshape(1, 256), w2s, beta2.reshape(1, 128),
                         Wc3, bc3.reshape(1, 2))
    return logits
